# baseline (speedup 1.0000x reference)
import functools

import jax
import jax.numpy as jnp
from jax import lax
from jax.experimental import pallas as pl
from jax.experimental.pallas import tpu as pltpu
from jax.experimental.pallas import tpu_sc as plsc

NC = 2
NS = 16
LANES = 16

_MESH = dict(core_axis_name="c", subcore_axis_name="s")
_SC_PARAMS = pltpu.CompilerParams(use_tc_tiling_on_sc=False,
                                  needs_layout_passes=False)

_RSQRT_MAGIC = 0x5F3759DF


def _pick_chunk(ept, cap):
    best = None
    for c in range(8, cap + 1, 8):
        if ept % c == 0:
            best = c
    return best


def _vec_rsqrt(d):
    y = plsc.bitcast(_RSQRT_MAGIC - (plsc.bitcast(d, jnp.int32) >> 1),
                     jnp.float32)
    half_d = 0.5 * d
    for _ in range(3):
        y = y * (1.5 - half_d * y * y)
    return y


def _sc_mega(xw2, src, dst3, b1c, n_pad, ch):
    e = src.shape[0]
    hw = xw2.shape[2]
    ept = e // NS
    n_ch = ept // ch
    slc = n_pad // NS
    half = slc // 2

    @functools.partial(
        pl.kernel,
        out_type=[
            jax.ShapeDtypeStruct((NC, n_pad, hw), jnp.float32),
            jax.ShapeDtypeStruct((NC * n_pad, hw), jnp.float32),
            jax.ShapeDtypeStruct((NC * n_pad, hw), jnp.float32),
        ],
        mesh=plsc.VectorSubcoreMesh(**_MESH),
        compiler_params=_SC_PARAMS,
        scratch_types=[
            pltpu.VMEM((ept,), jnp.int32),
            pltpu.VMEM((n_ch, ch), jnp.int32),
            pltpu.VMEM((ch, hw), jnp.float32),
            pltpu.VMEM((ch, hw), jnp.float32),
            pltpu.VMEM((half, hw), jnp.float32),
            pltpu.VMEM((slc, hw), jnp.float32),
            pltpu.VMEM((slc, hw), jnp.float32),
            pltpu.VMEM((slc,), jnp.float32),
            pltpu.VMEM((ch + 2 * LANES,), jnp.float32),
            pltpu.VMEM_SHARED((n_pad,), jnp.float32),
            pltpu.VMEM_SHARED((n_pad, hw), jnp.float32),
            pltpu.SemaphoreType.DMA,
            pltpu.SemaphoreType.DMA,
        ],
    )
    def mega(xw2_hbm, src_hbm, dst3_hbm, b1_hbm, z_hbm, t1_hbm, t2_hbm,
             si_v, di_v, rows0_v, rows1_v, stage_v, t_v, s_v,
             dinv_v, ones_v, deg_sh, acc_sh, sem0, sem1):
        c = lax.axis_index("c")
        s = lax.axis_index("s")
        rows = (rows0_v, rows1_v)
        sems = (sem0, sem1)
        nbase = s * slc
        tab0 = c * n_pad

        pltpu.sync_copy(src_hbm.at[pl.ds(s * ept, ept)], si_v)
        pltpu.sync_copy(dst3_hbm.at[s], di_v)
        @pl.loop(0, ch, step=LANES)
        def _(i):
            ones_v[pl.ds(i, LANES)] = jnp.full((LANES,), 1.0, jnp.float32)

        pltpu.sync_copy(b1_hbm.at[pl.ds(c * hw, hw)],
                        ones_v.at[pl.ds(ch, hw)])

        @pl.loop(0, ept, step=LANES)
        def _(i):
            si_v[pl.ds(i, LANES)] = si_v[pl.ds(i, LANES)] + tab0

        @pl.loop(0, slc, step=LANES)
        def _(i):
            dinv_v[pl.ds(i, LANES)] = jnp.zeros((LANES,), jnp.float32)

        pltpu.sync_copy(dinv_v, deg_sh.at[pl.ds(nbase, slc)])

        @pl.loop(0, half)
        def _(r):
            @pl.loop(0, hw, step=LANES)
            def _(f):
                stage_v[r, pl.ds(f, LANES)] = jnp.zeros((LANES,),
                                                        jnp.float32)

        pltpu.sync_copy(stage_v, acc_sh.at[pl.ds(nbase, half)])
        pltpu.sync_copy(stage_v, acc_sh.at[pl.ds(nbase + half, half)])
        plsc.subcore_barrier()

        @pl.loop(0, n_ch)
        def _(g):
            pltpu.sync_copy(ones_v.at[pl.ds(0, ch)], deg_sh.at[di_v.at[g]],
                            add=True)

        plsc.subcore_barrier()

        pltpu.sync_copy(deg_sh.at[pl.ds(nbase, slc)], dinv_v)

        @pl.loop(0, slc, step=LANES)
        def _(i):
            dinv_v[pl.ds(i, LANES)] = _vec_rsqrt(
                dinv_v[pl.ds(i, LANES)] + 1.0)

        pltpu.sync_copy(xw2_hbm.at[c, pl.ds(nbase, slc)], t_v)

        @pl.loop(0, slc)
        def _(r):
            bc = plsc.load_gather(
                dinv_v, [jnp.full((LANES,), r, jnp.int32)])

            @pl.loop(0, hw, step=LANES)
            def _(f):
                t_v[r, pl.ds(f, LANES)] = t_v[r, pl.ds(f, LANES)] * bc

        pltpu.sync_copy(t_v, t1_hbm.at[pl.ds(tab0 + nbase, slc)])
        pltpu.sync_copy(t1_hbm.at[pl.ds(tab0 + nbase + slc - 8, 8)],
                        s_v.at[pl.ds(0, 8)])
        plsc.subcore_barrier()

        def agg_ring(tab_hbm):
            for b in range(2):
                pltpu.async_copy(tab_hbm.at[si_v.at[pl.ds(b * ch, ch)]],
                                 rows[b], sems[b])

            @pl.loop(0, n_ch, step=2)
            def _(g0):
                for b in range(2):
                    g = g0 + b
                    pltpu.make_async_copy(
                        tab_hbm.at[si_v.at[pl.ds(0, ch)]],
                        rows[b], sems[b]).wait()
                    pltpu.sync_copy(rows[b], acc_sh.at[di_v.at[g]],
                                    add=True)

                    @pl.when(g + 2 < n_ch)
                    def _():
                        pltpu.async_copy(
                            tab_hbm.at[si_v.at[pl.ds((g + 2) * ch, ch)]],
                            rows[b], sems[b])

        agg_ring(t1_hbm)
        plsc.subcore_barrier()

        pltpu.sync_copy(acc_sh.at[pl.ds(nbase, slc)], s_v)

        @pl.loop(0, slc)
        def _(r):
            bc = plsc.load_gather(
                dinv_v, [jnp.full((LANES,), r, jnp.int32)])
            ha = bc * (s_v[r, pl.ds(0, LANES)] +
                       t_v[r, pl.ds(0, LANES)]) + ones_v[pl.ds(ch, LANES)]
            hb = (bc * (s_v[r, pl.ds(LANES, LANES)] +
                        t_v[r, pl.ds(LANES, LANES)])
                  + ones_v[pl.ds(ch + LANES, LANES)])
            t_v[r, pl.ds(0, LANES)] = bc * jnp.maximum(ha, 0.0)
            t_v[r, pl.ds(LANES, LANES)] = bc * jnp.maximum(hb, 0.0)

        pltpu.sync_copy(t_v, t2_hbm.at[pl.ds(tab0 + nbase, slc)])
        pltpu.sync_copy(t2_hbm.at[pl.ds(tab0 + nbase + slc - 8, 8)],
                        s_v.at[pl.ds(0, 8)])
        pltpu.sync_copy(stage_v, acc_sh.at[pl.ds(nbase, half)])
        pltpu.sync_copy(stage_v, acc_sh.at[pl.ds(nbase + half, half)])
        plsc.subcore_barrier()

        agg_ring(t2_hbm)
        plsc.subcore_barrier()

        pltpu.sync_copy(acc_sh.at[pl.ds(nbase, slc)], s_v)

        @pl.loop(0, slc)
        def _(r):
            bc = plsc.load_gather(
                dinv_v, [jnp.full((LANES,), r, jnp.int32)])

            @pl.loop(0, hw, step=LANES)
            def _(f):
                s_v[r, pl.ds(f, LANES)] = bc * (
                    s_v[r, pl.ds(f, LANES)] + t_v[r, pl.ds(f, LANES)])

        pltpu.sync_copy(s_v, z_hbm.at[c, pl.ds(nbase, slc)])

    return mega(xw2, src, dst3, b1c)


def _tc_matmul(a, w):
    n, k = a.shape
    m = w.shape[1]
    rb = 1000

    def body(a_ref, w_ref, o_ref):
        o_ref[...] = jnp.dot(a_ref[...], w_ref[...],
                             preferred_element_type=jnp.float32)

    return pl.pallas_call(
        body,
        grid=(n // rb,),
        in_specs=[pl.BlockSpec((rb, k), lambda i: (i, 0)),
                  pl.BlockSpec((k, m), lambda i: (0, 0))],
        out_specs=pl.BlockSpec((rb, m), lambda i: (i, 0)),
        out_shape=jax.ShapeDtypeStruct((n, m), jnp.float32),
    )(a, w)


def _tc_final(z0, z1, w2a, w2b, b2):
    n, hw = z0.shape
    m = w2a.shape[1]
    rb = 1000

    def body(z0_ref, z1_ref, wa_ref, wb_ref, b_ref, o_ref):
        o_ref[...] = (
            jnp.dot(z0_ref[...], wa_ref[...],
                    preferred_element_type=jnp.float32)
            + jnp.dot(z1_ref[...], wb_ref[...],
                      preferred_element_type=jnp.float32)
            + b_ref[...])

    return pl.pallas_call(
        body,
        grid=(n // rb,),
        in_specs=[pl.BlockSpec((rb, hw), lambda i: (i, 0)),
                  pl.BlockSpec((rb, hw), lambda i: (i, 0)),
                  pl.BlockSpec((hw, m), lambda i: (0, 0)),
                  pl.BlockSpec((hw, m), lambda i: (0, 0)),
                  pl.BlockSpec((1, m), lambda i: (0, 0))],
        out_specs=pl.BlockSpec((rb, m), lambda i: (i, 0)),
        out_shape=jax.ShapeDtypeStruct((n, m), jnp.float32),
    )(z0, z1, w2a, w2b, b2)


def kernel(x, edge_index, W1, b1, W2, b2):
    n = x.shape[0]
    e = edge_index.shape[1]
    d_hid = W1.shape[1]
    d_out = W2.shape[1]
    hw = d_hid // NC

    assert e % NS == 0 and d_hid % (NC * LANES) == 0
    ept = e // NS
    ch = _pick_chunk(ept, 200)
    n_ch = ept // ch
    n_pad = ((n + NS * 16 - 1) // (NS * 16)) * (NS * 16)

    src = edge_index[0]
    dst3 = edge_index[1].reshape(NS, n_ch, ch)

    xw = _tc_matmul(x, W1)
    xw2 = jnp.zeros((NC, n_pad, hw), jnp.float32)
    xw2 = xw2.at[0, :n].set(xw[:, :hw]).at[1, :n].set(xw[:, hw:])
    z, _, _ = _sc_mega(xw2, src, dst3, b1, n_pad, ch)

    out = _tc_final(z[0, :n], z[1, :n], W2[:hw], W2[hw:],
                    b2.reshape(1, d_out))
    return out


# speedup vs baseline: 1.0217x; 1.0217x over previous
import functools

import jax
import jax.numpy as jnp
from jax import lax
from jax.experimental import pallas as pl
from jax.experimental.pallas import tpu as pltpu
from jax.experimental.pallas import tpu_sc as plsc

NC = 2
NS = 16
LANES = 16

_MESH = dict(core_axis_name="c", subcore_axis_name="s")
_SC_PARAMS = pltpu.CompilerParams(use_tc_tiling_on_sc=False,
                                  needs_layout_passes=False)

_RSQRT_MAGIC = 0x5F3759DF


def _pick_chunk(ept, cap):
    best = None
    for c in range(8, cap + 1, 8):
        if ept % c == 0:
            best = c
    return best


def _vec_rsqrt(d):
    y = plsc.bitcast(_RSQRT_MAGIC - (plsc.bitcast(d, jnp.int32) >> 1),
                     jnp.float32)
    half_d = 0.5 * d
    for _ in range(3):
        y = y * (1.5 - half_d * y * y)
    return y


def _sc_mega(xw0, xw1, src, dst, b1c, n_pad, ch):
    e = src.shape[0]
    hw = xw0.shape[1]
    ept = e // NS
    n_ch = ept // ch
    chd = _pick_chunk(ept, 2000)
    n_chd = ept // chd
    slc = n_pad // NS
    half = slc // 2

    @functools.partial(
        pl.kernel,
        out_type=[
            jax.ShapeDtypeStruct((NC, n_pad, hw), jnp.float32),
            jax.ShapeDtypeStruct((NC * n_pad, hw), jnp.float32),
            jax.ShapeDtypeStruct((NC * n_pad, hw), jnp.float32),
        ],
        mesh=plsc.VectorSubcoreMesh(**_MESH),
        compiler_params=_SC_PARAMS,
        scratch_types=[
            pltpu.VMEM((ept,), jnp.int32),
            pltpu.VMEM((n_ch, ch), jnp.int32),
            pltpu.VMEM((ch, hw), jnp.float32),
            pltpu.VMEM((ch, hw), jnp.float32),
            pltpu.VMEM((half, hw), jnp.float32),
            pltpu.VMEM((slc, hw), jnp.float32),
            pltpu.VMEM((slc, hw), jnp.float32),
            pltpu.VMEM((slc,), jnp.float32),
            pltpu.VMEM((chd + 2 * LANES,), jnp.float32),
            pltpu.VMEM((chd,), jnp.int32),
            pltpu.VMEM_SHARED((n_pad,), jnp.float32),
            pltpu.VMEM_SHARED((n_pad, hw), jnp.float32),
            pltpu.SemaphoreType.DMA,
            pltpu.SemaphoreType.DMA,
        ],
    )
    def mega(xw0_hbm, xw1_hbm, src_hbm, dstf_hbm, b1_hbm,
             z_hbm, t1_hbm, t2_hbm,
             si_v, di_v, rows0_v, rows1_v, stage_v, t_v, s_v,
             dinv_v, ones_v, dchunk_v, deg_sh, acc_sh, sem0, sem1):
        c = lax.axis_index("c")
        s = lax.axis_index("s")
        rows = (rows0_v, rows1_v)
        sems = (sem0, sem1)
        nbase = s * slc
        tab0 = c * n_pad

        pltpu.sync_copy(src_hbm.at[pl.ds(s * ept, ept)], si_v)

        @pl.loop(0, n_ch)
        def _(g):
            pltpu.async_copy(dstf_hbm.at[pl.ds(s * ept + g * ch, ch)],
                             di_v.at[g], sem0)

        @pl.loop(0, n_ch)
        def _(g):
            pltpu.make_async_copy(dstf_hbm.at[pl.ds(s * ept, ch)],
                                  di_v.at[g], sem0).wait()
        @pl.loop(0, chd, step=LANES)
        def _(i):
            ones_v[pl.ds(i, LANES)] = jnp.full((LANES,), 1.0, jnp.float32)

        pltpu.sync_copy(b1_hbm.at[pl.ds(c * hw, hw)],
                        ones_v.at[pl.ds(chd, hw)])

        @pl.loop(0, ept, step=LANES)
        def _(i):
            si_v[pl.ds(i, LANES)] = si_v[pl.ds(i, LANES)] + tab0

        @pl.loop(0, slc, step=LANES)
        def _(i):
            dinv_v[pl.ds(i, LANES)] = jnp.zeros((LANES,), jnp.float32)

        pltpu.sync_copy(dinv_v, deg_sh.at[pl.ds(nbase, slc)])

        @pl.loop(0, half)
        def _(r):
            @pl.loop(0, hw, step=LANES)
            def _(f):
                stage_v[r, pl.ds(f, LANES)] = jnp.zeros((LANES,),
                                                        jnp.float32)

        pltpu.sync_copy(stage_v, acc_sh.at[pl.ds(nbase, half)])
        pltpu.sync_copy(stage_v, acc_sh.at[pl.ds(nbase + half, half)])
        plsc.subcore_barrier()

        @pl.loop(0, n_chd)
        def _(g):
            pltpu.sync_copy(dstf_hbm.at[pl.ds(s * ept + g * chd, chd)],
                            dchunk_v)
            pltpu.sync_copy(ones_v.at[pl.ds(0, chd)],
                            deg_sh.at[dchunk_v], add=True)

        plsc.subcore_barrier()

        pltpu.sync_copy(deg_sh.at[pl.ds(nbase, slc)], dinv_v)

        @pl.loop(0, slc, step=LANES)
        def _(i):
            dinv_v[pl.ds(i, LANES)] = _vec_rsqrt(
                dinv_v[pl.ds(i, LANES)] + 1.0)

        @pl.when(c == 0)
        def _():
            pltpu.sync_copy(xw0_hbm.at[pl.ds(nbase, slc)], t_v)

        @pl.when(c == 1)
        def _():
            pltpu.sync_copy(xw1_hbm.at[pl.ds(nbase, slc)], t_v)

        @pl.loop(0, slc)
        def _(r):
            bc = plsc.load_gather(
                dinv_v, [jnp.full((LANES,), r, jnp.int32)])

            @pl.loop(0, hw, step=LANES)
            def _(f):
                t_v[r, pl.ds(f, LANES)] = t_v[r, pl.ds(f, LANES)] * bc

        pltpu.sync_copy(t_v, t1_hbm.at[pl.ds(tab0 + nbase, slc)])
        pltpu.sync_copy(t1_hbm.at[pl.ds(tab0 + nbase + slc - 8, 8)],
                        s_v.at[pl.ds(0, 8)])
        plsc.subcore_barrier()

        def agg_ring(tab_hbm):
            for b in range(2):
                pltpu.async_copy(tab_hbm.at[si_v.at[pl.ds(b * ch, ch)]],
                                 rows[b], sems[b])

            @pl.loop(0, n_ch, step=2)
            def _(g0):
                for b in range(2):
                    g = g0 + b
                    pltpu.make_async_copy(
                        tab_hbm.at[si_v.at[pl.ds(0, ch)]],
                        rows[b], sems[b]).wait()
                    pltpu.sync_copy(rows[b], acc_sh.at[di_v.at[g]],
                                    add=True)

                    @pl.when(g + 2 < n_ch)
                    def _():
                        pltpu.async_copy(
                            tab_hbm.at[si_v.at[pl.ds((g + 2) * ch, ch)]],
                            rows[b], sems[b])

        agg_ring(t1_hbm)
        plsc.subcore_barrier()

        pltpu.sync_copy(acc_sh.at[pl.ds(nbase, slc)], s_v)

        @pl.loop(0, slc)
        def _(r):
            bc = plsc.load_gather(
                dinv_v, [jnp.full((LANES,), r, jnp.int32)])
            ha = bc * (s_v[r, pl.ds(0, LANES)] +
                       t_v[r, pl.ds(0, LANES)]) + ones_v[pl.ds(chd, LANES)]
            hb = (bc * (s_v[r, pl.ds(LANES, LANES)] +
                        t_v[r, pl.ds(LANES, LANES)])
                  + ones_v[pl.ds(chd + LANES, LANES)])
            t_v[r, pl.ds(0, LANES)] = bc * jnp.maximum(ha, 0.0)
            t_v[r, pl.ds(LANES, LANES)] = bc * jnp.maximum(hb, 0.0)

        pltpu.sync_copy(t_v, t2_hbm.at[pl.ds(tab0 + nbase, slc)])
        pltpu.sync_copy(t2_hbm.at[pl.ds(tab0 + nbase + slc - 8, 8)],
                        s_v.at[pl.ds(0, 8)])
        pltpu.sync_copy(stage_v, acc_sh.at[pl.ds(nbase, half)])
        pltpu.sync_copy(stage_v, acc_sh.at[pl.ds(nbase + half, half)])
        plsc.subcore_barrier()

        agg_ring(t2_hbm)
        plsc.subcore_barrier()

        pltpu.sync_copy(acc_sh.at[pl.ds(nbase, slc)], s_v)

        @pl.loop(0, slc)
        def _(r):
            bc = plsc.load_gather(
                dinv_v, [jnp.full((LANES,), r, jnp.int32)])

            @pl.loop(0, hw, step=LANES)
            def _(f):
                s_v[r, pl.ds(f, LANES)] = bc * (
                    s_v[r, pl.ds(f, LANES)] + t_v[r, pl.ds(f, LANES)])

        pltpu.sync_copy(s_v, z_hbm.at[c, pl.ds(nbase, slc)])

    return mega(xw0, xw1, src, dst, b1c)


def _tc_matmul_split(a, w, n_pad):
    n, k = a.shape
    hw = w.shape[1] // NC
    rb = 1000

    def body(a_ref, w_ref, o0_ref, o1_ref):
        xw = jnp.dot(a_ref[...], w_ref[...],
                     preferred_element_type=jnp.float32)
        o0_ref[...] = xw[:, :hw]
        o1_ref[...] = xw[:, hw:]

    return pl.pallas_call(
        body,
        grid=(n // rb,),
        in_specs=[pl.BlockSpec((rb, k), lambda i: (i, 0)),
                  pl.BlockSpec((k, NC * hw), lambda i: (0, 0))],
        out_specs=[pl.BlockSpec((rb, hw), lambda i: (i, 0)),
                   pl.BlockSpec((rb, hw), lambda i: (i, 0))],
        out_shape=[jax.ShapeDtypeStruct((n_pad, hw), jnp.float32),
                   jax.ShapeDtypeStruct((n_pad, hw), jnp.float32)],
    )(a, w)


def _tc_final(z0, z1, w2a, w2b, b2):
    n, hw = z0.shape
    m = w2a.shape[1]
    rb = 1000

    def body(z0_ref, z1_ref, wa_ref, wb_ref, b_ref, o_ref):
        o_ref[...] = (
            jnp.dot(z0_ref[...], wa_ref[...],
                    preferred_element_type=jnp.float32)
            + jnp.dot(z1_ref[...], wb_ref[...],
                      preferred_element_type=jnp.float32)
            + b_ref[...])

    return pl.pallas_call(
        body,
        grid=(n // rb,),
        in_specs=[pl.BlockSpec((rb, hw), lambda i: (i, 0)),
                  pl.BlockSpec((rb, hw), lambda i: (i, 0)),
                  pl.BlockSpec((hw, m), lambda i: (0, 0)),
                  pl.BlockSpec((hw, m), lambda i: (0, 0)),
                  pl.BlockSpec((1, m), lambda i: (0, 0))],
        out_specs=pl.BlockSpec((rb, m), lambda i: (i, 0)),
        out_shape=jax.ShapeDtypeStruct((n, m), jnp.float32),
    )(z0, z1, w2a, w2b, b2)


def kernel(x, edge_index, W1, b1, W2, b2):
    n = x.shape[0]
    e = edge_index.shape[1]
    d_hid = W1.shape[1]
    d_out = W2.shape[1]
    hw = d_hid // NC

    assert e % NS == 0 and d_hid % (NC * LANES) == 0
    ept = e // NS
    ch = _pick_chunk(ept, 200)
    n_ch = ept // ch
    n_pad = ((n + NS * 16 - 1) // (NS * 16)) * (NS * 16)

    src = edge_index[0]
    dst = edge_index[1]

    xw0, xw1 = _tc_matmul_split(x, W1, n_pad)
    z, _, _ = _sc_mega(xw0, xw1, src, dst, b1, n_pad, ch)

    out = _tc_final(z[0, :n], z[1, :n], W2[:hw], W2[hw:],
                    b2.reshape(1, d_out))
    return out


# speedup vs baseline: 1.0615x; 1.0389x over previous
import functools

import jax
import jax.numpy as jnp
from jax import lax
from jax.experimental import pallas as pl
from jax.experimental.pallas import tpu as pltpu
from jax.experimental.pallas import tpu_sc as plsc

NC = 2
NS = 16
LANES = 16

_MESH = dict(core_axis_name="c", subcore_axis_name="s")
_SC_PARAMS = pltpu.CompilerParams(use_tc_tiling_on_sc=False)


def _pick_chunk(ept, cap):
    best = None
    for c in range(8, cap + 1, 8):
        if ept % c == 0:
            best = c
    return best


def _sc_degree(dst, n_pad, ch):
    e = dst.shape[0]
    tiles = NC * NS
    ept = e // tiles
    slc = n_pad // NS

    @functools.partial(
        pl.kernel,
        out_type=jax.ShapeDtypeStruct((NC * n_pad,), jnp.float32),
        mesh=plsc.VectorSubcoreMesh(**_MESH),
        compiler_params=_SC_PARAMS,
        scratch_types=[
            pltpu.VMEM((ch,), jnp.int32),
            pltpu.VMEM((ch,), jnp.float32),
            pltpu.VMEM((slc,), jnp.float32),
            pltpu.VMEM_SHARED((n_pad,), jnp.float32),
            pltpu.SemaphoreType.DMA,
        ],
    )
    def deg_kernel(dst_hbm, out_hbm, idx_v, ones_v, zero_v, acc_sh, sem):
        c = lax.axis_index("c")
        s = lax.axis_index("s")
        wid = c * NS + s

        @pl.loop(0, ch, step=LANES)
        def _(i):
            ones_v[pl.ds(i, LANES)] = jnp.full((LANES,), 1.0, jnp.float32)

        @pl.loop(0, slc, step=LANES)
        def _(i):
            zero_v[pl.ds(i, LANES)] = jnp.zeros((LANES,), jnp.float32)

        pltpu.sync_copy(zero_v, acc_sh.at[pl.ds(s * slc, slc)])
        plsc.subcore_barrier()

        @pl.loop(0, ept, step=ch)
        def _(g):
            pltpu.sync_copy(dst_hbm.at[pl.ds(wid * ept + g, ch)], idx_v)
            pltpu.sync_copy(ones_v, acc_sh.at[idx_v], add=True)

        plsc.subcore_barrier()
        pltpu.sync_copy(acc_sh.at[pl.ds(s * slc, slc)], zero_v)
        pltpu.sync_copy(zero_v, out_hbm.at[pl.ds(c * n_pad + s * slc, slc)])

    return deg_kernel(dst)


def _sc_aggregate(table, src, dst, n_pad, ch):
    e = src.shape[0]
    d = table.shape[1]
    tiles = NC * NS
    ept = e // tiles
    slc = n_pad // NS
    n_ch = ept // ch

    @functools.partial(
        pl.kernel,
        out_type=jax.ShapeDtypeStruct((NC, n_pad, d), jnp.float32),
        mesh=plsc.VectorSubcoreMesh(**_MESH),
        compiler_params=_SC_PARAMS,
        scratch_types=[
            pltpu.VMEM((ept,), jnp.int32),
            pltpu.VMEM((n_ch, ch), jnp.int32),
            pltpu.VMEM((ch, d), jnp.float32),
            pltpu.VMEM((ch, d), jnp.float32),
            pltpu.VMEM((slc // 2, d), jnp.float32),
            pltpu.VMEM_SHARED((n_pad, d), jnp.float32),
            pltpu.SemaphoreType.DMA,
            pltpu.SemaphoreType.DMA,
        ],
    )
    def agg_kernel(t_hbm, src_hbm, dst3_hbm, out_hbm,
                   si_v, di_v, rows0_v, rows1_v, stage_v, acc_sh, sem0, sem1):
        c = lax.axis_index("c")
        s = lax.axis_index("s")
        wid = c * NS + s
        rows = (rows0_v, rows1_v)
        sems = (sem0, sem1)

        pltpu.sync_copy(src_hbm.at[pl.ds(wid * ept, ept)], si_v)
        pltpu.sync_copy(dst3_hbm.at[wid], di_v)

        half = slc // 2

        @pl.loop(0, half)
        def _(r):
            @pl.loop(0, d, step=LANES)
            def _(f):
                stage_v[r, pl.ds(f, LANES)] = jnp.zeros((LANES,), jnp.float32)

        pltpu.sync_copy(stage_v, acc_sh.at[pl.ds(s * slc, half)])
        pltpu.sync_copy(stage_v, acc_sh.at[pl.ds(s * slc + half, half)])
        plsc.subcore_barrier()

        for b in range(2):
            pltpu.async_copy(t_hbm.at[si_v.at[pl.ds(b * ch, ch)]],
                             rows[b], sems[b])

        @pl.loop(0, n_ch, step=2)
        def _(g0):
            for b in range(2):
                g = g0 + b
                pltpu.make_async_copy(
                    t_hbm.at[si_v.at[pl.ds(0, ch)]], rows[b], sems[b]).wait()
                pltpu.sync_copy(rows[b], acc_sh.at[di_v.at[g]], add=True)

                @pl.when(g + 2 < n_ch)
                def _():
                    pltpu.async_copy(
                        t_hbm.at[si_v.at[pl.ds((g + 2) * ch, ch)]],
                        rows[b], sems[b])

        plsc.subcore_barrier()
        for k in range(2):
            pltpu.sync_copy(acc_sh.at[pl.ds(s * slc + k * half, half)],
                            stage_v)
            pltpu.sync_copy(stage_v,
                            out_hbm.at[c, pl.ds(s * slc + k * half, half)])

    return agg_kernel(table, src, dst.reshape(tiles, n_ch, ch))


def _tc_matmul(a, w):
    n, k = a.shape
    m = w.shape[1]
    rb = 1000

    def body(a_ref, w_ref, o_ref):
        o_ref[...] = jnp.dot(a_ref[...], w_ref[...],
                             preferred_element_type=jnp.float32)

    return pl.pallas_call(
        body,
        grid=(n // rb,),
        in_specs=[pl.BlockSpec((rb, k), lambda i: (i, 0)),
                  pl.BlockSpec((k, m), lambda i: (0, 0))],
        out_specs=pl.BlockSpec((rb, m), lambda i: (i, 0)),
        out_shape=jax.ShapeDtypeStruct((n, m), jnp.float32),
    )(a, w)


def _tc_scale(xw, d0, d1):
    n, m = xw.shape
    rb = 1000

    def body(xw_ref, d0_ref, d1_ref, t_ref, dinv_ref):
        deg = d0_ref[...] + d1_ref[...] + 1.0
        dinv = lax.rsqrt(jnp.maximum(deg, 1.0))
        dinv_ref[...] = dinv
        t_ref[...] = xw_ref[...] * dinv

    return pl.pallas_call(
        body,
        grid=(n // rb,),
        in_specs=[pl.BlockSpec((rb, m), lambda i: (i, 0)),
                  pl.BlockSpec((rb, 1), lambda i: (i, 0)),
                  pl.BlockSpec((rb, 1), lambda i: (i, 0))],
        out_specs=[pl.BlockSpec((rb, m), lambda i: (i, 0)),
                   pl.BlockSpec((rb, 1), lambda i: (i, 0))],
        out_shape=[jax.ShapeDtypeStruct((n, m), jnp.float32),
                   jax.ShapeDtypeStruct((n, 1), jnp.float32)],
    )(xw, d0, d1)


def _tc_relu_scale(s_a, s_b, t, dinv, b):
    n, m = t.shape
    rb = 1000

    def body(sa_ref, sb_ref, t_ref, dinv_ref, b_ref, o_ref):
        dinv = dinv_ref[...]
        h = dinv * (sa_ref[...] + sb_ref[...] + t_ref[...]) + b_ref[...]
        o_ref[...] = dinv * jnp.maximum(h, 0.0)

    return pl.pallas_call(
        body,
        grid=(n // rb,),
        in_specs=[pl.BlockSpec((rb, m), lambda i: (i, 0)),
                  pl.BlockSpec((rb, m), lambda i: (i, 0)),
                  pl.BlockSpec((rb, m), lambda i: (i, 0)),
                  pl.BlockSpec((rb, 1), lambda i: (i, 0)),
                  pl.BlockSpec((1, m), lambda i: (0, 0))],
        out_specs=pl.BlockSpec((rb, m), lambda i: (i, 0)),
        out_shape=jax.ShapeDtypeStruct((n, m), jnp.float32),
    )(s_a, s_b, t, dinv, b)


def _tc_final(s_a, s_b, t, dinv, w2, b2):
    n, k = t.shape
    m = w2.shape[1]
    rb = 1000

    def body(sa_ref, sb_ref, t_ref, dinv_ref, w_ref, b_ref, o_ref):
        z = dinv_ref[...] * (sa_ref[...] + sb_ref[...] + t_ref[...])
        o_ref[...] = jnp.dot(z, w_ref[...],
                             preferred_element_type=jnp.float32) + b_ref[...]

    return pl.pallas_call(
        body,
        grid=(n // rb,),
        in_specs=[pl.BlockSpec((rb, k), lambda i: (i, 0)),
                  pl.BlockSpec((rb, k), lambda i: (i, 0)),
                  pl.BlockSpec((rb, k), lambda i: (i, 0)),
                  pl.BlockSpec((rb, 1), lambda i: (i, 0)),
                  pl.BlockSpec((k, m), lambda i: (0, 0)),
                  pl.BlockSpec((1, m), lambda i: (0, 0))],
        out_specs=pl.BlockSpec((rb, m), lambda i: (i, 0)),
        out_shape=jax.ShapeDtypeStruct((n, m), jnp.float32),
    )(s_a, s_b, t, dinv, w2, b2)


def kernel(x, edge_index, W1, b1, W2, b2):
    n = x.shape[0]
    e = edge_index.shape[1]
    d_hid = W1.shape[1]

    tiles = NC * NS
    assert e % tiles == 0
    ept = e // tiles
    ch_deg = _pick_chunk(ept, 2000)
    ch_agg = _pick_chunk(ept, 200)
    n_pad = ((n + NS * 8 - 1) // (NS * 8)) * (NS * 8)

    src = edge_index[0]
    dst = edge_index[1]

    deg_parts = _sc_degree(dst, n_pad, ch_deg)
    xw = _tc_matmul(x, W1)

    d0 = deg_parts[:n].reshape(n, 1)
    d1 = deg_parts[n_pad:n_pad + n].reshape(n, 1)
    t1, dinv = _tc_scale(xw, d0, d1)

    s1 = _sc_aggregate(t1, src, dst, n_pad, ch_agg)
    t2 = _tc_relu_scale(s1[0, :n], s1[1, :n], t1, dinv,
                        b1.reshape(1, d_hid))

    s2 = _sc_aggregate(t2, src, dst, n_pad, ch_agg)
    out = _tc_final(s2[0, :n], s2[1, :n], t2, dinv, W2,
                    b2.reshape(1, W2.shape[1]))
    return out
